# baseline (speedup 1.0000x reference)
import jax
import jax.numpy as jnp
from jax import lax
from jax.experimental import pallas as pl
from jax.experimental.pallas import tpu as pltpu
from jax.experimental.pallas import tpu_sc as plsc

NC = 2
NS = 16
NW = NC * NS

VOCAB = 100000
DIM = 1024
NTOK = 8192
ROWS_PER_W = NTOK // NW
G = 32
NG = ROWS_PER_W // G
NV = DIM // 16


def _body(idx_hbm, t0, t1, t2, o0, o1, o2, o3, o4, o5,
          idx_v, widx, bufa, bufb, obufa, obufb, sema, semb, osema, osemb):
    wid = lax.axis_index("s") * NC + lax.axis_index("c")
    b = wid // 8
    s0 = (wid % 8) * ROWS_PER_W
    pltpu.sync_copy(idx_hbm.at[b, pl.ds(s0, ROWS_PER_W)], idx_v)
    for i in range(ROWS_PER_W // 16):
        widx[pl.ds(i * 16, 16)] = idx_v[pl.ds(i * 16, 16)] >> 1

    ti = (t0.bitcast(jnp.int32), t1.bitcast(jnp.int32), t2.bitcast(jnp.int32))
    oi = (o0.bitcast(jnp.int32), o1.bitcast(jnp.int32), o2.bitcast(jnp.int32))
    oi2 = (o3.bitcast(jnp.int32), o4.bitcast(jnp.int32), o5.bitcast(jnp.int32))
    obase = wid * (ROWS_PER_W // 2)
    bufs = (bufa, bufb)
    obufs = (obufa, obufb)
    sems = (sema, semb)
    osems = (osema, osemb)

    NS_TOT = 3 * NG

    def gidx(g):
        return widx.at[pl.ds(pl.multiple_of(g * G, G), G)]

    def orows(g):
        return pl.ds(pl.multiple_of(obase + g * (G // 2), G // 2), G // 2)

    def fire(s, slot):
        t = s // NG
        g = s - t * NG
        for tt in range(3):
            @pl.when(t == tt)
            def _():
                pltpu.async_copy(ti[tt].at[gidx(g)], bufs[slot], sems[slot])

    def step(s, slot):
        t = s // NG
        g = s - t * NG
        buf = bufs[slot]
        obuf = obufs[slot]
        pltpu.make_async_copy(ti[0].at[gidx(g)], buf, sems[slot]).wait()

        @pl.when(s >= 2)
        def _():
            pltpu.make_async_copy(obuf, oi[0].at[pl.ds(0, G // 2)], osems[slot]).wait()
            pltpu.make_async_copy(obuf, oi[0].at[pl.ds(0, G // 2)], osems[slot]).wait()

        base_tok = pl.multiple_of(g * G, G)
        veca = idx_v[pl.ds(base_tok, 16)]
        vecb = idx_v[pl.ds(base_tok + 16, 16)]
        for k in range(G // 2):
            vec = veca if k < 8 else vecb
            kk = k % 8
            sh0 = (vec[2 * kk] & 1) * 16
            sh1 = (vec[2 * kk + 1] & 1) * 16

            @pl.loop(0, NV // 8)
            def _(c):
                for u in range(8):
                    col = pl.ds((c * 8 + u) * 16, 16)
                    a = buf[2 * k, col]
                    bb = buf[2 * k + 1, col]
                    lo = (a >> sh0) & jnp.int32(0xFFFF)
                    obuf[k, col] = lo | ((bb >> sh1) << 16)

        for tt in range(3):
            @pl.when(t == tt)
            def _():
                pltpu.async_copy(obuf, oi[tt].at[orows(g)], osems[slot])
                pltpu.async_copy(obuf, oi2[tt].at[orows(g)], osems[slot])

        @pl.when(s + 2 < NS_TOT)
        def _():
            fire(s + 2, slot)

    fire(jnp.int32(0), 0)
    fire(jnp.int32(1), 1)

    @pl.loop(0, NS_TOT // 2)
    def _(h):
        step(2 * h, 0)
        step(2 * h + 1, 1)

    for slot in (0, 1):
        pltpu.make_async_copy(obufs[slot], oi[0].at[pl.ds(0, G // 2)], osems[slot]).wait()
        pltpu.make_async_copy(obufs[slot], oi[0].at[pl.ds(0, G // 2)], osems[slot]).wait()


@jax.jit
def _gather3(idx, table0, table1, table2):
    mesh = plsc.VectorSubcoreMesh(core_axis_name="c", subcore_axis_name="s")
    out = jax.ShapeDtypeStruct((NTOK, DIM), jnp.bfloat16)
    return pl.kernel(
        _body,
        out_type=(out, out, out, out, out, out),
        mesh=mesh,
        scratch_types=[
            pltpu.VMEM((ROWS_PER_W,), jnp.int32),
            pltpu.VMEM((ROWS_PER_W,), jnp.int32),
            pltpu.VMEM((G, DIM), jnp.int32),
            pltpu.VMEM((G, DIM), jnp.int32),
            pltpu.VMEM((G // 2, DIM), jnp.int32),
            pltpu.VMEM((G // 2, DIM), jnp.int32),
            pltpu.SemaphoreType.DMA,
            pltpu.SemaphoreType.DMA,
            pltpu.SemaphoreType.DMA,
            pltpu.SemaphoreType.DMA,
        ],
    )(idx, table0, table1, table2)


def kernel(inputs, table0, table1, table2):
    B, S = inputs.shape
    outs = _gather3(inputs, table0, table1, table2)
    return tuple(o.reshape(B, S, DIM) for o in outs)


# speedup vs baseline: 1.0212x; 1.0212x over previous
import jax
import jax.numpy as jnp
from jax import lax
from jax.experimental import pallas as pl
from jax.experimental.pallas import tpu as pltpu
from jax.experimental.pallas import tpu_sc as plsc

NC = 2
NS = 16
NW = NC * NS

VOCAB = 100000
DIM = 1024
NTOK = 8192
ROWS_PER_W = NTOK // NW
G = 16
NG = ROWS_PER_W // G
NV = DIM // 16


NSLOT = 4


def _body(idx_hbm, t0, t1, t2, o0, o1, o2, o3, o4, o5,
          idx_v, widx, buf0, buf1, buf2, buf3, obuf0, obuf1, obuf2, obuf3,
          sem0, sem1, sem2, sem3, osem0, osem1, osem2, osem3):
    wid = lax.axis_index("s") * NC + lax.axis_index("c")
    b = wid // 8
    s0 = (wid % 8) * ROWS_PER_W
    pltpu.sync_copy(idx_hbm.at[b, pl.ds(s0, ROWS_PER_W)], idx_v)
    for i in range(ROWS_PER_W // 16):
        widx[pl.ds(i * 16, 16)] = idx_v[pl.ds(i * 16, 16)] >> 1

    ti = (t0.bitcast(jnp.int32), t1.bitcast(jnp.int32), t2.bitcast(jnp.int32))
    oi = (o0.bitcast(jnp.int32), o1.bitcast(jnp.int32), o2.bitcast(jnp.int32))
    oi2 = (o3.bitcast(jnp.int32), o4.bitcast(jnp.int32), o5.bitcast(jnp.int32))
    obase = wid * (ROWS_PER_W // 2)
    bufs = (buf0, buf1, buf2, buf3)
    obufs = (obuf0, obuf1, obuf2, obuf3)
    sems = (sem0, sem1, sem2, sem3)
    osems = (osem0, osem1, osem2, osem3)

    NS_TOT = 3 * NG

    def gidx(g):
        return widx.at[pl.ds(pl.multiple_of(g * G, G), G)]

    def orows(g):
        return pl.ds(pl.multiple_of(obase + g * (G // 2), G // 2), G // 2)

    def fire(s, slot):
        t = s // NG
        g = s - t * NG
        for tt in range(3):
            @pl.when(t == tt)
            def _():
                pltpu.async_copy(ti[tt].at[gidx(g)], bufs[slot], sems[slot])

    def step(s, slot):
        t = s // NG
        g = s - t * NG
        buf = bufs[slot]
        obuf = obufs[slot]
        pltpu.make_async_copy(ti[0].at[gidx(g)], buf, sems[slot]).wait()

        @pl.when(s + 2 < NS_TOT)
        def _():
            fire(s + 2, (slot + 2) % NSLOT)

        @pl.when(s >= NSLOT)
        def _():
            pltpu.make_async_copy(obuf, oi[0].at[pl.ds(0, G // 2)], osems[slot]).wait()
            pltpu.make_async_copy(obuf, oi[0].at[pl.ds(0, G // 2)], osems[slot]).wait()

        vec = idx_v[pl.ds(pl.multiple_of(g * G, G), G)]
        for k in range(G // 2):
            sh0 = (vec[2 * k] & 1) * 16
            sh1 = (vec[2 * k + 1] & 1) * 16

            @pl.loop(0, NV // 8)
            def _(c):
                for u in range(8):
                    col = pl.ds((c * 8 + u) * 16, 16)
                    a = buf[2 * k, col]
                    bb = buf[2 * k + 1, col]
                    lo = (a >> sh0) & jnp.int32(0xFFFF)
                    obuf[k, col] = lo | ((bb >> sh1) << 16)

        for tt in range(3):
            @pl.when(t == tt)
            def _():
                pltpu.async_copy(obuf, oi[tt].at[orows(g)], osems[slot])
                pltpu.async_copy(obuf, oi2[tt].at[orows(g)], osems[slot])

    fire(jnp.int32(0), 0)
    fire(jnp.int32(1), 1)

    @pl.loop(0, NS_TOT // NSLOT)
    def _(h):
        for j in range(NSLOT):
            step(NSLOT * h + j, j)

    for slot in range(NSLOT):
        pltpu.make_async_copy(obufs[slot], oi[0].at[pl.ds(0, G // 2)], osems[slot]).wait()
        pltpu.make_async_copy(obufs[slot], oi[0].at[pl.ds(0, G // 2)], osems[slot]).wait()


@jax.jit
def _gather3(idx, table0, table1, table2):
    mesh = plsc.VectorSubcoreMesh(core_axis_name="c", subcore_axis_name="s")
    out = jax.ShapeDtypeStruct((NTOK, DIM), jnp.bfloat16)
    return pl.kernel(
        _body,
        out_type=(out, out, out, out, out, out),
        mesh=mesh,
        scratch_types=[
            pltpu.VMEM((ROWS_PER_W,), jnp.int32),
            pltpu.VMEM((ROWS_PER_W,), jnp.int32),
            pltpu.VMEM((G, DIM), jnp.int32),
            pltpu.VMEM((G, DIM), jnp.int32),
            pltpu.VMEM((G, DIM), jnp.int32),
            pltpu.VMEM((G, DIM), jnp.int32),
            pltpu.VMEM((G // 2, DIM), jnp.int32),
            pltpu.VMEM((G // 2, DIM), jnp.int32),
            pltpu.VMEM((G // 2, DIM), jnp.int32),
            pltpu.VMEM((G // 2, DIM), jnp.int32),
            pltpu.SemaphoreType.DMA,
            pltpu.SemaphoreType.DMA,
            pltpu.SemaphoreType.DMA,
            pltpu.SemaphoreType.DMA,
            pltpu.SemaphoreType.DMA,
            pltpu.SemaphoreType.DMA,
            pltpu.SemaphoreType.DMA,
            pltpu.SemaphoreType.DMA,
        ],
    )(idx, table0, table1, table2)


def kernel(inputs, table0, table1, table2):
    B, S = inputs.shape
    outs = _gather3(inputs, table0, table1, table2)
    return tuple(o.reshape(B, S, DIM) for o in outs)


# speedup vs baseline: 1.0300x; 1.0086x over previous
import jax
import jax.numpy as jnp
from jax import lax
from jax.experimental import pallas as pl
from jax.experimental.pallas import tpu as pltpu
from jax.experimental.pallas import tpu_sc as plsc

NC = 2
NS = 16
NW = NC * NS

VOCAB = 100000
DIM = 1024
NTOK = 8192
ROWS_PER_W = NTOK // NW
G = 16
NG = ROWS_PER_W // G
NV = DIM // 16


def _body(idx_hbm, t0, t1, t2, o0, o1, o2, o3, o4, o5,
          idx_v, widx, bufa, bufb, obufa, obufb, sema, semb, osema, osemb):
    wid = lax.axis_index("s") * NC + lax.axis_index("c")
    b = wid // 8
    s0 = (wid % 8) * ROWS_PER_W
    pltpu.sync_copy(idx_hbm.at[b, pl.ds(s0, ROWS_PER_W)], idx_v)
    for i in range(ROWS_PER_W // 16):
        widx[pl.ds(i * 16, 16)] = idx_v[pl.ds(i * 16, 16)] >> 1

    ti = (t0.bitcast(jnp.int32), t1.bitcast(jnp.int32), t2.bitcast(jnp.int32))
    oi = (o0.bitcast(jnp.int32), o1.bitcast(jnp.int32), o2.bitcast(jnp.int32))
    oi2 = (o3.bitcast(jnp.int32), o4.bitcast(jnp.int32), o5.bitcast(jnp.int32))
    obase = wid * (ROWS_PER_W // 2)
    bufs = (bufa, bufb)
    obufs = (obufa, obufb)
    sems = (sema, semb)
    osems = (osema, osemb)

    NS_TOT = 3 * NG

    def gidx(g):
        return widx.at[pl.ds(pl.multiple_of(g * G, G), G)]

    def orows(g):
        return pl.ds(pl.multiple_of(obase + g * (G // 2), G // 2), G // 2)

    def fire(s, slot):
        t = s // NG
        g = s - t * NG
        for tt in range(3):
            @pl.when(t == tt)
            def _():
                pltpu.async_copy(ti[tt].at[gidx(g)], bufs[slot], sems[slot])

    def step(s, slot):
        t = s // NG
        g = s - t * NG
        buf = bufs[slot]
        obuf = obufs[slot]
        pltpu.make_async_copy(ti[0].at[gidx(g)], buf, sems[slot]).wait()

        @pl.when(s >= 2)
        def _():
            pltpu.make_async_copy(obuf, oi[0].at[pl.ds(0, G // 2)], osems[slot]).wait()
            pltpu.make_async_copy(obuf, oi[0].at[pl.ds(0, G // 2)], osems[slot]).wait()

        vec = idx_v[pl.ds(pl.multiple_of(g * G, G), G)]
        shifts = [(vec[j] & 1) * 16 for j in range(G)]

        @pl.loop(0, NV // 8)
        def _(c):
            for u in range(8):
                col = pl.ds((c * 8 + u) * 16, 16)
                for k in range(G // 2):
                    a = buf[2 * k, col]
                    bb = buf[2 * k + 1, col]
                    lo = (a >> shifts[2 * k]) & jnp.int32(0xFFFF)
                    obuf[k, col] = lo | ((bb >> shifts[2 * k + 1]) << 16)

        for tt in range(3):
            @pl.when(t == tt)
            def _():
                pltpu.async_copy(obuf, oi[tt].at[orows(g)], osems[slot])
                pltpu.async_copy(obuf, oi2[tt].at[orows(g)], osems[slot])

        @pl.when(s + 2 < NS_TOT)
        def _():
            fire(s + 2, slot)

    fire(jnp.int32(0), 0)
    fire(jnp.int32(1), 1)

    @pl.loop(0, NS_TOT // 2)
    def _(h):
        step(2 * h, 0)
        step(2 * h + 1, 1)

    for slot in (0, 1):
        pltpu.make_async_copy(obufs[slot], oi[0].at[pl.ds(0, G // 2)], osems[slot]).wait()
        pltpu.make_async_copy(obufs[slot], oi[0].at[pl.ds(0, G // 2)], osems[slot]).wait()


@jax.jit
def _gather3(idx, table0, table1, table2):
    mesh = plsc.VectorSubcoreMesh(core_axis_name="c", subcore_axis_name="s")
    out = jax.ShapeDtypeStruct((NTOK, DIM), jnp.bfloat16)
    return pl.kernel(
        _body,
        out_type=(out, out, out, out, out, out),
        mesh=mesh,
        scratch_types=[
            pltpu.VMEM((ROWS_PER_W,), jnp.int32),
            pltpu.VMEM((ROWS_PER_W,), jnp.int32),
            pltpu.VMEM((G, DIM), jnp.int32),
            pltpu.VMEM((G, DIM), jnp.int32),
            pltpu.VMEM((G // 2, DIM), jnp.int32),
            pltpu.VMEM((G // 2, DIM), jnp.int32),
            pltpu.SemaphoreType.DMA,
            pltpu.SemaphoreType.DMA,
            pltpu.SemaphoreType.DMA,
            pltpu.SemaphoreType.DMA,
        ],
    )(idx, table0, table1, table2)


def kernel(inputs, table0, table1, table2):
    B, S = inputs.shape
    outs = _gather3(inputs, table0, table1, table2)
    return tuple(o.reshape(B, S, DIM) for o in outs)


# speedup vs baseline: 1.0351x; 1.0050x over previous
import jax
import jax.numpy as jnp
from jax import lax
from jax.experimental import pallas as pl
from jax.experimental.pallas import tpu as pltpu
from jax.experimental.pallas import tpu_sc as plsc

NC = 2
NS = 16
NW = NC * NS

VOCAB = 100000
DIM = 1024
NTOK = 8192
ROWS_PER_W = NTOK // NW
G = 16
NG = ROWS_PER_W // G
NV = DIM // 16


def _body(idx_hbm, t0, t1, t2, o0, o1, o2, o3, o4, o5,
          idx_v, widx, bufa, bufb, obufa, obufb, sema, semb, osema, osemb):
    wid = lax.axis_index("s") * NC + lax.axis_index("c")
    b = wid // 8
    s0 = (wid % 8) * ROWS_PER_W
    pltpu.sync_copy(idx_hbm.at[b, pl.ds(s0, ROWS_PER_W)], idx_v)
    for i in range(ROWS_PER_W // 16):
        widx[pl.ds(i * 16, 16)] = idx_v[pl.ds(i * 16, 16)] >> 1

    ti = (t0.bitcast(jnp.int32), t1.bitcast(jnp.int32), t2.bitcast(jnp.int32))
    oi = (o0.bitcast(jnp.int32), o1.bitcast(jnp.int32), o2.bitcast(jnp.int32))
    oi2 = (o3.bitcast(jnp.int32), o4.bitcast(jnp.int32), o5.bitcast(jnp.int32))
    obase = wid * (ROWS_PER_W // 2)
    bufs = (bufa, bufb)
    obufs = (obufa, obufb)
    sems = (sema, semb)
    osems = (osema, osemb)

    NS_TOT = 3 * NG

    def gidx(g):
        return widx.at[pl.ds(pl.multiple_of(g * G, G), G)]

    def orows(g):
        return pl.ds(pl.multiple_of(obase + g * (G // 2), G // 2), G // 2)

    def fire(s, slot):
        t = s // NG
        g = s - t * NG
        for tt in range(3):
            @pl.when(t == tt)
            def _():
                pltpu.async_copy(ti[tt].at[gidx(g)], bufs[slot], sems[slot])

    def step(s, slot):
        t = s // NG
        g = s - t * NG
        buf = bufs[slot]
        obuf = obufs[slot]
        pltpu.make_async_copy(ti[0].at[gidx(g)], buf, sems[slot]).wait()

        @pl.when(s >= 2)
        def _():
            pltpu.make_async_copy(obuf, oi[0].at[pl.ds(0, G // 2)], osems[slot]).wait()
            pltpu.make_async_copy(obuf, oi[0].at[pl.ds(0, G // 2)], osems[slot]).wait()

        vec = idx_v[pl.ds(pl.multiple_of(g * G, G), G)]
        for k in range(G // 2):
            sh0 = (vec[2 * k] & 1) * 16
            sh1 = (vec[2 * k + 1] & 1) * 16

            @pl.loop(0, NV // 8)
            def _(c):
                for u in range(8):
                    col = pl.ds((c * 8 + u) * 16, 16)
                    a = buf[2 * k, col]
                    bb = buf[2 * k + 1, col]
                    lo = (a >> sh0) & jnp.int32(0xFFFF)
                    obuf[k, col] = lo | ((bb >> sh1) << 16)

        @pl.when(s + 2 < NS_TOT)
        def _():
            fire(s + 2, slot)

        for tt in range(3):
            @pl.when(t == tt)
            def _():
                pltpu.async_copy(obuf, oi[tt].at[orows(g)], osems[slot])
                pltpu.async_copy(obuf, oi2[tt].at[orows(g)], osems[slot])

    fire(jnp.int32(0), 0)
    fire(jnp.int32(1), 1)

    @pl.loop(0, NS_TOT // 2)
    def _(h):
        step(2 * h, 0)
        step(2 * h + 1, 1)

    for slot in (0, 1):
        pltpu.make_async_copy(obufs[slot], oi[0].at[pl.ds(0, G // 2)], osems[slot]).wait()
        pltpu.make_async_copy(obufs[slot], oi[0].at[pl.ds(0, G // 2)], osems[slot]).wait()


@jax.jit
def _gather3(idx, table0, table1, table2):
    mesh = plsc.VectorSubcoreMesh(core_axis_name="c", subcore_axis_name="s")
    out = jax.ShapeDtypeStruct((NTOK, DIM), jnp.bfloat16)
    return pl.kernel(
        _body,
        out_type=(out, out, out, out, out, out),
        mesh=mesh,
        scratch_types=[
            pltpu.VMEM((ROWS_PER_W,), jnp.int32),
            pltpu.VMEM((ROWS_PER_W,), jnp.int32),
            pltpu.VMEM((G, DIM), jnp.int32),
            pltpu.VMEM((G, DIM), jnp.int32),
            pltpu.VMEM((G // 2, DIM), jnp.int32),
            pltpu.VMEM((G // 2, DIM), jnp.int32),
            pltpu.SemaphoreType.DMA,
            pltpu.SemaphoreType.DMA,
            pltpu.SemaphoreType.DMA,
            pltpu.SemaphoreType.DMA,
        ],
    )(idx, table0, table1, table2)


def kernel(inputs, table0, table1, table2):
    B, S = inputs.shape
    outs = _gather3(inputs, table0, table1, table2)
    return tuple(o.reshape(B, S, DIM) for o in outs)
